# baseline (speedup 1.0000x reference)
import functools

import jax
import jax.numpy as jnp
from jax import lax
from jax.experimental import pallas as pl
from jax.experimental.pallas import tpu as pltpu
from jax.experimental.pallas import tpu_sc as plsc

N_DATASETS = 3
NUM_USERS = 100000
DIM = 64
BATCH = 16384

NUM_CORES = 2
NUM_SUBCORES = 16
NUM_WORKERS = NUM_CORES * NUM_SUBCORES
N_COLS = N_DATASETS * DIM
COLS_PER_W = N_COLS // NUM_WORKERS
LANES = 16

SPLIT = 49920
NB = NUM_USERS - SPLIT
QTR = BATCH // 4
QITER = QTR // LANES


def _sc_gather(table_t, idx_flat):
  mesh = plsc.VectorSubcoreMesh(core_axis_name="c", subcore_axis_name="s")

  @functools.partial(
      pl.kernel,
      out_type=jax.ShapeDtypeStruct((N_COLS, BATCH), jnp.float32),
      mesh=mesh,
      scratch_types=[
          pltpu.VMEM((BATCH,), jnp.int32),
          pltpu.VMEM((SPLIT,), jnp.float32),
          pltpu.VMEM((NB,), jnp.float32),
          pltpu.VMEM((QTR,), jnp.float32),
          pltpu.VMEM((QTR,), jnp.float32),
          pltpu.SemaphoreType.DMA,
          pltpu.SemaphoreType.DMA,
          pltpu.SemaphoreType.DMA,
          pltpu.SemaphoreType.DMA,
      ],
      compiler_params=pltpu.CompilerParams(
          use_tc_tiling_on_sc=True, needs_layout_passes=False),
  )
  def k(tab_hbm, idx_hbm, out_hbm, idx_v, bufa_v, bufb_v, out0_v, out1_v,
        sema, semb, semi, semo):
    outs = [out0_v, out1_v]
    wid = lax.axis_index("s") * NUM_CORES + lax.axis_index("c")

    def half_a(j):
      col = wid * COLS_PER_W + j
      d = col // DIM
      f = col - d * DIM
      return pltpu.make_async_copy(
          tab_hbm.at[d, f, pl.ds(0, SPLIT)], bufa_v, sema)

    def half_b(j):
      col = wid * COLS_PER_W + j
      d = col // DIM
      f = col - d * DIM
      return pltpu.make_async_copy(
          tab_hbm.at[d, f, pl.ds(SPLIT, NB)], bufb_v, semb)

    def pass_a(q, slot):
      with jax.named_scope("pass_a"):
        @plsc.parallel_loop(0, QITER, unroll=8)
        def body(v):
          u16 = idx_v[pl.ds(q * QTR + v * LANES, LANES)]
          ga = plsc.load_gather(bufa_v, [jnp.minimum(u16, SPLIT - 1)])
          outs[slot][pl.ds(v * LANES, LANES)] = jnp.where(
              u16 < SPLIT, ga, 0.0)

    def pass_b(q, slot):
      with jax.named_scope("pass_b"):
        @plsc.parallel_loop(0, QITER, unroll=8)
        def body(v):
          u16 = idx_v[pl.ds(q * QTR + v * LANES, LANES)]
          ub = jnp.minimum(jnp.maximum(u16 - SPLIT, 0), NB - 1)
          gb = plsc.load_gather(bufb_v, [ub])
          pos = lax.iota(jnp.int32, LANES) + (v * LANES)
          plsc.store_scatter(outs[slot], [pos], gb, mask=u16 >= SPLIT)

    cpi = pltpu.make_async_copy(idx_hbm, idx_v, semi)
    cpi.start()
    cpa = half_a(0)
    cpa.start()
    cpb = half_b(0)
    cpb.start()
    cpi.wait()

    out_cps = [None, None]

    def wait_slot(slot):
      if out_cps[slot] is not None:
        with jax.named_scope("out_wait"):
          out_cps[slot].wait()
        out_cps[slot] = None

    def wr(j, q, slot):
      col = wid * COLS_PER_W + j
      out_cps[slot] = pltpu.make_async_copy(
          outs[slot], out_hbm.at[col, pl.ds(q * QTR, QTR)], semo)
      out_cps[slot].start()

    for j in range(COLS_PER_W):
      with jax.named_scope("wait_a"):
        cpa.wait()
      wait_slot(0)
      pass_a(0, 0)
      wait_slot(1)
      pass_a(1, 1)
      with jax.named_scope("wait_b"):
        cpb.wait()
      pass_b(0, 0)
      wr(j, 0, 0)
      wait_slot(0)
      pass_a(2, 0)
      pass_b(1, 1)
      wr(j, 1, 1)
      wait_slot(1)
      pass_a(3, 1)
      if j + 1 < COLS_PER_W:
        cpa = half_a(j + 1)
        cpa.start()
      pass_b(2, 0)
      wr(j, 2, 0)
      pass_b(3, 1)
      if j + 1 < COLS_PER_W:
        cpb = half_b(j + 1)
        cpb.start()
      wr(j, 3, 1)

    for cp in out_cps:
      if cp is not None:
        cp.wait()

  return k(table_t, idx_flat)


def kernel(user_embeds_list, userIdx):
  table_t = jnp.transpose(user_embeds_list, (0, 2, 1))
  idx_flat = userIdx.astype(jnp.int32)
  out_t = _sc_gather(table_t, idx_flat)
  return jnp.transpose(out_t)


# speedup vs baseline: 1.2270x; 1.2270x over previous
import functools

import jax
import jax.numpy as jnp
from jax import lax
from jax.experimental import pallas as pl
from jax.experimental.pallas import tpu as pltpu
from jax.experimental.pallas import tpu_sc as plsc

N_DATASETS = 3
NUM_USERS = 100000
DIM = 64
BATCH = 16384

NUM_CORES = 2
NUM_SUBCORES = 16
NUM_WORKERS = NUM_CORES * NUM_SUBCORES
N_COLS = N_DATASETS * DIM
COLS_PER_W = N_COLS // NUM_WORKERS
LANES = 16
HALF = BATCH // 2


def _sc_gather(table_t, idx_flat):
  mesh = plsc.VectorSubcoreMesh(core_axis_name="c", subcore_axis_name="s")

  @functools.partial(
      pl.kernel,
      out_type=jax.ShapeDtypeStruct((N_COLS, BATCH), jnp.float32),
      mesh=mesh,
      scratch_types=[
          pltpu.VMEM((BATCH,), jnp.int32),
          pltpu.VMEM((NUM_USERS,), jnp.float32),
          pltpu.VMEM((HALF,), jnp.float32),
      ],
      compiler_params=pltpu.CompilerParams(
          use_tc_tiling_on_sc=True, needs_layout_passes=False),
  )
  def k(tab_hbm, idx_hbm, out_hbm, idx_v, row_v, out_v):
    wid = lax.axis_index("s") * NUM_CORES + lax.axis_index("c")
    pltpu.sync_copy(idx_hbm, idx_v)

    for j in range(COLS_PER_W):
      col = wid * COLS_PER_W + j
      d = col // DIM
      f = col - d * DIM
      pltpu.sync_copy(tab_hbm.at[d, f], row_v)

      for half in range(2):
        @plsc.parallel_loop(0, HALF // LANES, unroll=16)
        def body(v):
          u16 = idx_v[pl.ds(half * HALF + v * LANES, LANES)]
          out_v[pl.ds(v * LANES, LANES)] = plsc.load_gather(row_v, [u16])
        pltpu.sync_copy(out_v, out_hbm.at[col, pl.ds(half * HALF, HALF)])

  return k(table_t, idx_flat)


def kernel(user_embeds_list, userIdx):
  table_t = jnp.transpose(user_embeds_list, (0, 2, 1))
  idx_flat = userIdx.astype(jnp.int32)
  out_t = _sc_gather(table_t, idx_flat)
  return jnp.transpose(out_t)


# speedup vs baseline: 1.2335x; 1.0053x over previous
import functools

import jax
import jax.numpy as jnp
from jax import lax
from jax.experimental import pallas as pl
from jax.experimental.pallas import tpu as pltpu
from jax.experimental.pallas import tpu_sc as plsc

N_DATASETS = 3
NUM_USERS = 100000
DIM = 64
BATCH = 16384

NUM_CORES = 2
NUM_SUBCORES = 16
NUM_WORKERS = NUM_CORES * NUM_SUBCORES
N_COLS = N_DATASETS * DIM
COLS_PER_W = N_COLS // NUM_WORKERS
LANES = 16
HALF = BATCH // 2


def _sc_gather(table_t, idx_flat):
  mesh = plsc.VectorSubcoreMesh(core_axis_name="c", subcore_axis_name="s")

  @functools.partial(
      pl.kernel,
      out_type=jax.ShapeDtypeStruct((N_COLS, BATCH), jnp.float32),
      mesh=mesh,
      scratch_types=[
          pltpu.VMEM((BATCH,), jnp.int32),
          pltpu.VMEM((NUM_USERS,), jnp.float32),
          pltpu.VMEM((HALF,), jnp.float32),
      ],
      compiler_params=pltpu.CompilerParams(
          use_tc_tiling_on_sc=True, needs_layout_passes=False),
  )
  def k(tab_hbm, idx_hbm, out_hbm, idx_v, row_v, out_v):
    wid = lax.axis_index("s") * NUM_CORES + lax.axis_index("c")
    pltpu.sync_copy(idx_hbm, idx_v)

    for j in range(COLS_PER_W):
      col = wid * COLS_PER_W + j
      d = col // DIM
      f = col - d * DIM
      pltpu.sync_copy(tab_hbm.at[d, f], row_v)

      for half in range(2):
        @plsc.parallel_loop(0, HALF // LANES, unroll=8)
        def body(v):
          u16 = idx_v[pl.ds(half * HALF + v * LANES, LANES)]
          out_v[pl.ds(v * LANES, LANES)] = plsc.load_gather(row_v, [u16])
        pltpu.sync_copy(out_v, out_hbm.at[col, pl.ds(half * HALF, HALF)])

  return k(table_t, idx_flat)


def kernel(user_embeds_list, userIdx):
  table_t = jnp.transpose(user_embeds_list, (0, 2, 1))
  idx_flat = userIdx.astype(jnp.int32)
  out_t = _sc_gather(table_t, idx_flat)
  return jnp.transpose(out_t)
